# Initial kernel scaffold; baseline (speedup 1.0000x reference)
import jax
import jax.numpy as jnp
from jax.experimental import pallas as pl


def kernel(node_feat, node_config_feat, node_opcode, edge_index, batch, emb_op, emb_shape, W1, b1, pW0, pb0, Wl0, bl0, Wr0, pW1, pb1, Wl1, bl1, Wr1, pW2, pb2, Wl2, bl2, Wr2, post_W, post_b):
    raise NotImplementedError("write your pallas kernel here")


# speedup vs baseline: 4.6234x; 4.6234x over previous
import dataclasses
import functools

import jax
import jax.numpy as jnp
from jax import lax
from jax.experimental import pallas as pl
from jax.experimental.pallas import tpu as pltpu
from jax.experimental.pallas import tpu_sc as plsc

N = 10000
NPAD = 10240
G = 16
N_OPS = 120
NCF = 18
NFD = 140
SET = 8
SED = 4
OPD = 32
H = 64

NC = 2
NS = 16
NW = NC * NS
CH = 128
ROWS_PER_TILE = NPAD // NS

_f32 = jnp.float32


def _make_seg_kernel(e_pad, with_count):
    epw = e_pad // NW
    n_chunks = epw // CH
    mesh = plsc.VectorSubcoreMesh(core_axis_name="c", subcore_axis_name="s")

    out_types = [jax.ShapeDtypeStruct((NC, NPAD, H), _f32)]
    if with_count:
        out_types.append(jax.ShapeDtypeStruct((NW, NPAD), _f32))

    scratch = [
        pltpu.VMEM((CH,), jnp.int32),
        pltpu.VMEM((CH,), jnp.int32),
        pltpu.VMEM((CH, H), _f32),
        pltpu.VMEM_SHARED((NPAD, H), _f32),
        pltpu.SemaphoreType.DMA,
    ]
    if with_count:
        scratch.append(pltpu.VMEM((NPAD,), _f32))

    out_type = tuple(out_types) if with_count else out_types[0]

    cp = pltpu.CompilerParams()
    for fld, val in (("needs_layout_passes", False),
                     ("use_tc_tiling_on_sc", False)):
        if fld in pltpu.CompilerParams.__dataclass_fields__:
            cp = dataclasses.replace(cp, **{fld: val})

    @functools.partial(pl.kernel, out_type=out_type, mesh=mesh,
                       scratch_types=scratch, compiler_params=cp)
    def k(y_hbm, src_hbm, dst_hbm, zeros_hbm, *rest):
        if with_count:
            out_hbm, outc_hbm, src_v, dst_v, rows_v, acc_sh, sem, cnt_v = rest
        else:
            out_hbm, src_v, dst_v, rows_v, acc_sh, sem = rest
        cid = lax.axis_index("c")
        sid = lax.axis_index("s")
        wid = cid * NS + sid

        pltpu.sync_copy(zeros_hbm.at[pl.ds(sid * ROWS_PER_TILE, ROWS_PER_TILE)],
                        acc_sh.at[pl.ds(sid * ROWS_PER_TILE, ROWS_PER_TILE)])
        if with_count:
            zero16 = jnp.zeros((16,), _f32)

            @pl.loop(0, NPAD // 16)
            def _(j):
                cnt_v[pl.ds(j * 16, 16)] = zero16
        plsc.subcore_barrier()

        base = wid * epw

        @pl.loop(0, n_chunks)
        def _(i):
            off = base + i * CH
            pltpu.sync_copy(src_hbm.at[pl.ds(off, CH)], src_v)
            pltpu.sync_copy(dst_hbm.at[pl.ds(off, CH)], dst_v)
            pltpu.async_copy(y_hbm.at[src_v], rows_v, sem).wait()
            pltpu.sync_copy(rows_v, acc_sh.at[dst_v], add=True)
            if with_count:
                ones16 = jnp.ones((16,), _f32)
                for j in range(CH // 16):
                    idx = dst_v[pl.ds(j * 16, 16)]
                    plsc.addupdate_scatter(cnt_v, [idx], ones16)

        plsc.subcore_barrier()
        pltpu.sync_copy(acc_sh.at[pl.ds(sid * ROWS_PER_TILE, ROWS_PER_TILE)],
                        out_hbm.at[cid, pl.ds(sid * ROWS_PER_TILE, ROWS_PER_TILE)])
        if with_count:
            pltpu.sync_copy(cnt_v, outc_hbm.at[wid])

    return k


def _encode_body(nf_ref, ncf_ref, opc_ref, emb_op_ref, emb_sh_ref, W1_ref,
                 b1_ref, out_ref):
    nf = nf_ref[...]
    W1 = W1_ref[...]
    sidx = nf[:, NFD:NFD + 1].astype(jnp.int32)
    oh_sh = (sidx == lax.broadcasted_iota(jnp.int32, (1, SET), 1)).astype(_f32)
    opc = opc_ref[...]
    oh_op = (opc == lax.broadcasted_iota(jnp.int32, (1, N_OPS), 1)).astype(_f32)
    t_sh = jnp.dot(emb_sh_ref[...], W1[NFD:NFD + SED],
                   preferred_element_type=_f32)
    t_op = jnp.dot(emb_op_ref[...], W1[NFD + SED:NFD + SED + OPD],
                   preferred_element_type=_f32)
    acc = jnp.dot(nf[:, :NFD], W1[:NFD], preferred_element_type=_f32)
    acc = acc + jnp.dot(ncf_ref[...], W1[NFD + SED + OPD:],
                        preferred_element_type=_f32)
    acc = acc + jnp.dot(oh_sh, t_sh, preferred_element_type=_f32)
    acc = acc + jnp.dot(oh_op, t_op, preferred_element_type=_f32)
    out_ref[...] = jnp.maximum(acc + b1_ref[...], 0.0)


def _pre_body(x_ref, pW_ref, pb_ref, Wl_ref, Wr_ref, y_ref, r_ref):
    x = x_ref[...]
    xp = jnp.maximum(jnp.dot(x, pW_ref[...], preferred_element_type=_f32)
                     + pb_ref[...], 0.0)
    y_ref[...] = jnp.dot(xp, Wl_ref[...], preferred_element_type=_f32)
    r_ref[...] = jnp.dot(x, Wr_ref[...], preferred_element_type=_f32)


def _post0_body(s_ref, cntp_ref, r_ref, bl_ref, x_ref, cnt_ref):
    cnt = lax.dot_general(cntp_ref[...], jnp.ones((NW, 1), _f32),
                          (((0,), (0,)), ((), ())),
                          preferred_element_type=_f32)
    cnt_ref[...] = cnt
    s = s_ref[...]
    z = (s[0] + s[1]) / jnp.maximum(cnt, 1.0)
    out = z + bl_ref[...] + r_ref[...]
    nrm = jnp.sqrt(jnp.sum(out * out, axis=1, keepdims=True))
    x_ref[...] = out / jnp.maximum(nrm, 1e-12)


def _post_body(s_ref, cnt_ref, r_ref, bl_ref, x_ref):
    s = s_ref[...]
    z = (s[0] + s[1]) / jnp.maximum(cnt_ref[...], 1.0)
    out = z + bl_ref[...] + r_ref[...]
    nrm = jnp.sqrt(jnp.sum(out * out, axis=1, keepdims=True))
    x_ref[...] = out / jnp.maximum(nrm, 1e-12)


def _pool_body(x_ref, batch_ref, pW_ref, pb_ref, out_ref):
    x = x_ref[...]
    b = batch_ref[...]
    oh = (b == lax.broadcasted_iota(jnp.int32, (1, G), 1)).astype(_f32)
    xsum = jax.lax.dot_general(oh, x, (((0,), (0,)), ((), ())),
                               preferred_element_type=_f32)
    gcnt = lax.dot_general(oh, jnp.ones((x.shape[0], 1), _f32),
                           (((0,), (0,)), ((), ())),
                           preferred_element_type=_f32)
    neg = jnp.float32(-jnp.inf)
    rows = []
    for g in range(G):
        m = jnp.max(jnp.where(oh[:, g:g + 1] > 0, x, neg), axis=0,
                    keepdims=True)
        rows.append(m)
    xmax = jnp.concatenate(rows, axis=0)
    xg = xmax + xsum / jnp.maximum(gcnt, 1.0)
    xg = xg / jnp.sqrt(jnp.sum(xg * xg, axis=1, keepdims=True))
    out_ref[...] = jnp.dot(xg, pW_ref[...], preferred_element_type=_f32) \
        + pb_ref[...]


def _tc_call(body, out_shapes):
    return pl.pallas_call(body, out_shape=out_shapes)


def kernel(node_feat, node_config_feat, node_opcode, edge_index, batch,
           emb_op, emb_shape, W1, b1,
           pW0, pb0, Wl0, bl0, Wr0,
           pW1, pb1, Wl1, bl1, Wr1,
           pW2, pb2, Wl2, bl2, Wr2,
           post_W, post_b):
    n = node_feat.shape[0]
    e = edge_index.shape[1]
    src = edge_index[0].astype(jnp.int32)
    dst = edge_index[1].astype(jnp.int32)
    e_pad = -(-e // (NW * CH)) * (NW * CH)
    pad = e_pad - e
    srcp = jnp.concatenate([src, jnp.zeros((pad,), jnp.int32)])
    dstp = jnp.concatenate([dst, jnp.full((pad,), n, jnp.int32)])
    zeros_pad = jnp.zeros((NPAD, H), _f32)

    opc = node_opcode.astype(jnp.int32).reshape(n, 1)
    bat = batch.astype(jnp.int32).reshape(n, 1)

    sds = jax.ShapeDtypeStruct
    x0 = _tc_call(_encode_body, sds((n, 2 * H), _f32))(
        node_feat, node_config_feat, opc, emb_op, emb_shape, W1,
        b1.reshape(1, -1))

    seg0 = _make_seg_kernel(e_pad, True)
    seg = _make_seg_kernel(e_pad, False)
    pre = _tc_call(_pre_body, [sds((n, H), _f32), sds((n, H), _f32)])
    post0 = _tc_call(_post0_body, [sds((n, H), _f32), sds((n, 1), _f32)])
    post = _tc_call(_post_body, sds((n, H), _f32))

    y, r = pre(x0, pW0, pb0.reshape(1, -1), Wl0, Wr0)
    s, cntp = seg0(y, srcp, dstp, zeros_pad)
    x, cnt = post0(s[:, :n], cntp[:, :n], r, bl0.reshape(1, -1))
    y, r = pre(x, pW1, pb1.reshape(1, -1), Wl1, Wr1)
    s = seg(y, srcp, dstp, zeros_pad)
    x = post(s[:, :n], cnt, r, bl1.reshape(1, -1))
    y, r = pre(x, pW2, pb2.reshape(1, -1), Wl2, Wr2)
    s = seg(y, srcp, dstp, zeros_pad)
    x = post(s[:, :n], cnt, r, bl2.reshape(1, -1))

    out = _tc_call(_pool_body, sds((G, 1), _f32))(
        x, bat, post_W, post_b.reshape(1, -1))
    return (out, out)


# speedup vs baseline: 4.8521x; 1.0495x over previous
import dataclasses
import functools

import jax
import jax.numpy as jnp
from jax import lax
from jax.experimental import pallas as pl
from jax.experimental.pallas import tpu as pltpu
from jax.experimental.pallas import tpu_sc as plsc

N = 10000
NPAD = 10240
G = 16
N_OPS = 120
NCF = 18
NFD = 140
SET = 8
SED = 4
OPD = 32
H = 64

NC = 2
NS = 16
NW = NC * NS
CH = 128
ROWS_PER_TILE = NPAD // NS

_f32 = jnp.float32


K = 8


def _make_seg_kernel(e_pad, with_count):
    epw = e_pad // NW
    n_super = epw // (CH * K)
    mesh = plsc.VectorSubcoreMesh(core_axis_name="c", subcore_axis_name="s")

    out_types = [jax.ShapeDtypeStruct((NC, NPAD, H), _f32)]
    if with_count:
        out_types.append(jax.ShapeDtypeStruct((NW, NPAD), _f32))

    scratch = []
    scratch += [pltpu.VMEM((CH,), jnp.int32) for _ in range(K)]
    scratch += [pltpu.VMEM((CH,), jnp.int32) for _ in range(K)]
    scratch += [pltpu.VMEM((CH, H), _f32) for _ in range(K)]
    scratch += [
        pltpu.VMEM_SHARED((NPAD, H), _f32),
        pltpu.SemaphoreType.DMA,
        pltpu.SemaphoreType.DMA,
        pltpu.SemaphoreType.DMA,
    ]
    if with_count:
        scratch.append(pltpu.VMEM((NPAD,), _f32))

    out_type = tuple(out_types) if with_count else out_types[0]

    cp = pltpu.CompilerParams()
    for fld, val in (("needs_layout_passes", False),
                     ("use_tc_tiling_on_sc", False)):
        if fld in pltpu.CompilerParams.__dataclass_fields__:
            cp = dataclasses.replace(cp, **{fld: val})

    @functools.partial(pl.kernel, out_type=out_type, mesh=mesh,
                       scratch_types=scratch, compiler_params=cp)
    def k(y_hbm, src_hbm, dst_hbm, zeros_hbm, *rest):
        rest = list(rest)
        out_hbm = rest.pop(0)
        if with_count:
            outc_hbm = rest.pop(0)
        src_v = [rest.pop(0) for _ in range(K)]
        dst_v = [rest.pop(0) for _ in range(K)]
        rows_v = [rest.pop(0) for _ in range(K)]
        acc_sh, sem_i, sem_g, sem_s = rest[:4]
        if with_count:
            cnt_v = rest[4]
        cid = lax.axis_index("c")
        sid = lax.axis_index("s")
        wid = cid * NS + sid

        pltpu.sync_copy(zeros_hbm.at[pl.ds(sid * ROWS_PER_TILE, ROWS_PER_TILE)],
                        acc_sh.at[pl.ds(sid * ROWS_PER_TILE, ROWS_PER_TILE)])
        if with_count:
            zero16 = jnp.zeros((16,), _f32)

            @pl.loop(0, NPAD // 16)
            def _(j):
                cnt_v[pl.ds(j * 16, 16)] = zero16
        plsc.subcore_barrier()

        base = wid * epw

        @pl.loop(0, n_super)
        def _(s):
            off = base + s * (CH * K)
            hs = []
            for j in range(K):
                hs.append(pltpu.async_copy(
                    src_hbm.at[pl.ds(off + j * CH, CH)], src_v[j], sem_i))
                hs.append(pltpu.async_copy(
                    dst_hbm.at[pl.ds(off + j * CH, CH)], dst_v[j], sem_i))
            for h in hs:
                h.wait()
            hs = [pltpu.async_copy(y_hbm.at[src_v[j]], rows_v[j], sem_g)
                  for j in range(K)]
            for h in hs:
                h.wait()
            hs = [pltpu.async_copy(rows_v[j], acc_sh.at[dst_v[j]], sem_s,
                                   add=True)
                  for j in range(K)]
            if with_count:
                ones16 = jnp.ones((16,), _f32)
                for j in range(K):
                    for t in range(CH // 16):
                        idx = dst_v[j][pl.ds(t * 16, 16)]
                        plsc.addupdate_scatter(cnt_v, [idx], ones16)
            for h in hs:
                h.wait()

        plsc.subcore_barrier()
        pltpu.sync_copy(acc_sh.at[pl.ds(sid * ROWS_PER_TILE, ROWS_PER_TILE)],
                        out_hbm.at[cid, pl.ds(sid * ROWS_PER_TILE, ROWS_PER_TILE)])
        if with_count:
            pltpu.sync_copy(cnt_v, outc_hbm.at[wid])

    return k


def _encode_body(nf_ref, ncf_ref, opc_ref, emb_op_ref, emb_sh_ref, W1_ref,
                 b1_ref, out_ref):
    nf = nf_ref[...]
    W1 = W1_ref[...]
    sidx = nf[:, NFD:NFD + 1].astype(jnp.int32)
    oh_sh = (sidx == lax.broadcasted_iota(jnp.int32, (1, SET), 1)).astype(_f32)
    opc = opc_ref[...]
    oh_op = (opc == lax.broadcasted_iota(jnp.int32, (1, N_OPS), 1)).astype(_f32)
    t_sh = jnp.dot(emb_sh_ref[...], W1[NFD:NFD + SED],
                   preferred_element_type=_f32)
    t_op = jnp.dot(emb_op_ref[...], W1[NFD + SED:NFD + SED + OPD],
                   preferred_element_type=_f32)
    acc = jnp.dot(nf[:, :NFD], W1[:NFD], preferred_element_type=_f32)
    acc = acc + jnp.dot(ncf_ref[...], W1[NFD + SED + OPD:],
                        preferred_element_type=_f32)
    acc = acc + jnp.dot(oh_sh, t_sh, preferred_element_type=_f32)
    acc = acc + jnp.dot(oh_op, t_op, preferred_element_type=_f32)
    out_ref[...] = jnp.maximum(acc + b1_ref[...], 0.0)


def _pre_body(x_ref, pW_ref, pb_ref, Wl_ref, Wr_ref, y_ref, r_ref):
    x = x_ref[...]
    xp = jnp.maximum(jnp.dot(x, pW_ref[...], preferred_element_type=_f32)
                     + pb_ref[...], 0.0)
    y_ref[...] = jnp.dot(xp, Wl_ref[...], preferred_element_type=_f32)
    r_ref[...] = jnp.dot(x, Wr_ref[...], preferred_element_type=_f32)


def _post0_body(s_ref, cntp_ref, r_ref, bl_ref, x_ref, cnt_ref):
    cnt = lax.dot_general(cntp_ref[...], jnp.ones((NW, 1), _f32),
                          (((0,), (0,)), ((), ())),
                          preferred_element_type=_f32)
    cnt_ref[...] = cnt
    s = s_ref[...]
    z = (s[0] + s[1]) / jnp.maximum(cnt, 1.0)
    out = z + bl_ref[...] + r_ref[...]
    nrm = jnp.sqrt(jnp.sum(out * out, axis=1, keepdims=True))
    x_ref[...] = out / jnp.maximum(nrm, 1e-12)


def _post_body(s_ref, cnt_ref, r_ref, bl_ref, x_ref):
    s = s_ref[...]
    z = (s[0] + s[1]) / jnp.maximum(cnt_ref[...], 1.0)
    out = z + bl_ref[...] + r_ref[...]
    nrm = jnp.sqrt(jnp.sum(out * out, axis=1, keepdims=True))
    x_ref[...] = out / jnp.maximum(nrm, 1e-12)


def _pool_body(x_ref, batch_ref, pW_ref, pb_ref, out_ref):
    x = x_ref[...]
    b = batch_ref[...]
    oh = (b == lax.broadcasted_iota(jnp.int32, (1, G), 1)).astype(_f32)
    xsum = jax.lax.dot_general(oh, x, (((0,), (0,)), ((), ())),
                               preferred_element_type=_f32)
    gcnt = lax.dot_general(oh, jnp.ones((x.shape[0], 1), _f32),
                           (((0,), (0,)), ((), ())),
                           preferred_element_type=_f32)
    neg = jnp.float32(-jnp.inf)
    rows = []
    for g in range(G):
        m = jnp.max(jnp.where(oh[:, g:g + 1] > 0, x, neg), axis=0,
                    keepdims=True)
        rows.append(m)
    xmax = jnp.concatenate(rows, axis=0)
    xg = xmax + xsum / jnp.maximum(gcnt, 1.0)
    xg = xg / jnp.sqrt(jnp.sum(xg * xg, axis=1, keepdims=True))
    out_ref[...] = jnp.dot(xg, pW_ref[...], preferred_element_type=_f32) \
        + pb_ref[...]


def _tc_call(body, out_shapes):
    return pl.pallas_call(body, out_shape=out_shapes)


def kernel(node_feat, node_config_feat, node_opcode, edge_index, batch,
           emb_op, emb_shape, W1, b1,
           pW0, pb0, Wl0, bl0, Wr0,
           pW1, pb1, Wl1, bl1, Wr1,
           pW2, pb2, Wl2, bl2, Wr2,
           post_W, post_b):
    n = node_feat.shape[0]
    e = edge_index.shape[1]
    src = edge_index[0].astype(jnp.int32)
    dst = edge_index[1].astype(jnp.int32)
    e_pad = -(-e // (NW * CH * K)) * (NW * CH * K)
    pad = e_pad - e
    srcp = jnp.concatenate([src, jnp.zeros((pad,), jnp.int32)])
    dstp = jnp.concatenate([dst, jnp.full((pad,), n, jnp.int32)])
    zeros_pad = jnp.zeros((NPAD, H), _f32)

    opc = node_opcode.astype(jnp.int32).reshape(n, 1)
    bat = batch.astype(jnp.int32).reshape(n, 1)

    sds = jax.ShapeDtypeStruct
    x0 = _tc_call(_encode_body, sds((n, 2 * H), _f32))(
        node_feat, node_config_feat, opc, emb_op, emb_shape, W1,
        b1.reshape(1, -1))

    seg0 = _make_seg_kernel(e_pad, True)
    seg = _make_seg_kernel(e_pad, False)
    pre = _tc_call(_pre_body, [sds((n, H), _f32), sds((n, H), _f32)])
    post0 = _tc_call(_post0_body, [sds((n, H), _f32), sds((n, 1), _f32)])
    post = _tc_call(_post_body, sds((n, H), _f32))

    y, r = pre(x0, pW0, pb0.reshape(1, -1), Wl0, Wr0)
    s, cntp = seg0(y, srcp, dstp, zeros_pad)
    x, cnt = post0(s[:, :n], cntp[:, :n], r, bl0.reshape(1, -1))
    y, r = pre(x, pW1, pb1.reshape(1, -1), Wl1, Wr1)
    s = seg(y, srcp, dstp, zeros_pad)
    x = post(s[:, :n], cnt, r, bl1.reshape(1, -1))
    y, r = pre(x, pW2, pb2.reshape(1, -1), Wl2, Wr2)
    s = seg(y, srcp, dstp, zeros_pad)
    x = post(s[:, :n], cnt, r, bl2.reshape(1, -1))

    out = _tc_call(_pool_body, sds((G, 1), _f32))(
        x, bat, post_W, post_b.reshape(1, -1))
    return (out, out)


# speedup vs baseline: 9.6953x; 1.9982x over previous
import dataclasses
import functools

import jax
import jax.numpy as jnp
from jax import lax
from jax.experimental import pallas as pl
from jax.experimental.pallas import tpu as pltpu
from jax.experimental.pallas import tpu_sc as plsc

N = 10000
NPAD = 10240
G = 16
N_OPS = 120
NCF = 18
NFD = 140
SET = 8
SED = 4
OPD = 32
H = 64

NC = 2
NS = 16
NW = NC * NS
CH = 128
ROWS_PER_TILE = NPAD // NS

_f32 = jnp.float32


K = 8


HW = H // NC


def _make_seg_kernel(e_pad, with_count):
    epw = e_pad // NS
    n_super = epw // (CH * K)
    mesh = plsc.VectorSubcoreMesh(core_axis_name="c", subcore_axis_name="s")

    out_types = [jax.ShapeDtypeStruct((NC, NPAD, HW), _f32)]
    if with_count:
        out_types.append(jax.ShapeDtypeStruct((NW, NPAD), _f32))

    scratch = []
    scratch += [pltpu.VMEM((CH,), jnp.int32) for _ in range(K)]
    scratch += [pltpu.VMEM((CH,), jnp.int32) for _ in range(K)]
    scratch += [pltpu.VMEM((CH, HW), _f32) for _ in range(K)]
    scratch += [
        pltpu.VMEM_SHARED((NPAD, HW), _f32),
        pltpu.VMEM_SHARED((NPAD, HW), _f32),
        pltpu.SemaphoreType.DMA,
        pltpu.SemaphoreType.DMA,
        pltpu.SemaphoreType.DMA,
    ]
    if with_count:
        scratch.append(pltpu.VMEM((NPAD,), _f32))

    out_type = tuple(out_types) if with_count else out_types[0]

    cp = pltpu.CompilerParams()
    for fld, val in (("needs_layout_passes", False),
                     ("use_tc_tiling_on_sc", False)):
        if fld in pltpu.CompilerParams.__dataclass_fields__:
            cp = dataclasses.replace(cp, **{fld: val})

    @functools.partial(pl.kernel, out_type=out_type, mesh=mesh,
                       scratch_types=scratch, compiler_params=cp)
    def k(y_hbm, src_hbm, dst_hbm, *rest):
        rest = list(rest)
        out_hbm = rest.pop(0)
        if with_count:
            outc_hbm = rest.pop(0)
        src_v = [rest.pop(0) for _ in range(K)]
        dst_v = [rest.pop(0) for _ in range(K)]
        rows_v = [rest.pop(0) for _ in range(K)]
        acc_sh, y_sh, sem_i, sem_g, sem_s = rest[:5]
        if with_count:
            cnt_v = rest[5]
        cid = lax.axis_index("c")
        sid = lax.axis_index("s")
        wid = cid * NS + sid

        zero16 = jnp.zeros((16,), _f32)

        @pl.loop(0, CH)
        def _(i):
            for c in range(HW // 16):
                rows_v[0][i, pl.ds(c * 16, 16)] = zero16

        stripe = pl.ds(sid * ROWS_PER_TILE, ROWS_PER_TILE)
        hs = [pltpu.async_copy(
                  rows_v[0],
                  acc_sh.at[pl.ds(sid * ROWS_PER_TILE + t * CH, CH)], sem_s)
              for t in range(ROWS_PER_TILE // CH)]
        hs.append(pltpu.async_copy(y_hbm.at[cid, stripe], y_sh.at[stripe],
                                   sem_g))
        for h in hs:
            h.wait()
        if with_count:
            @pl.loop(0, NPAD // 16)
            def _(j):
                cnt_v[pl.ds(j * 16, 16)] = zero16
        plsc.subcore_barrier()

        base = sid * epw

        @pl.loop(0, n_super)
        def _(s):
            off = base + s * (CH * K)
            hs = []
            for j in range(K):
                hs.append(pltpu.async_copy(
                    src_hbm.at[pl.ds(off + j * CH, CH)], src_v[j], sem_i))
                hs.append(pltpu.async_copy(
                    dst_hbm.at[pl.ds(off + j * CH, CH)], dst_v[j], sem_i))
            for h in hs:
                h.wait()
            hs = [pltpu.async_copy(y_sh.at[src_v[j]], rows_v[j], sem_g)
                  for j in range(K)]
            for h in hs:
                h.wait()
            hs = [pltpu.async_copy(rows_v[j], acc_sh.at[dst_v[j]], sem_s,
                                   add=True)
                  for j in range(K)]
            if with_count:
                @pl.when(lax.rem(s, 2) == cid)
                def _():
                    ones16 = jnp.ones((16,), _f32)
                    for j in range(K):
                        for t in range(CH // 16):
                            idx = dst_v[j][pl.ds(t * 16, 16)]
                            plsc.addupdate_scatter(cnt_v, [idx], ones16)
            for h in hs:
                h.wait()

        plsc.subcore_barrier()
        pltpu.sync_copy(acc_sh.at[pl.ds(sid * ROWS_PER_TILE, ROWS_PER_TILE)],
                        out_hbm.at[cid, pl.ds(sid * ROWS_PER_TILE, ROWS_PER_TILE)])
        if with_count:
            pltpu.sync_copy(cnt_v, outc_hbm.at[wid])

    return k


def _encode_body(nf_ref, ncf_ref, opc_ref, emb_op_ref, emb_sh_ref, W1_ref,
                 b1_ref, out_ref):
    nf = nf_ref[...]
    W1 = W1_ref[...]
    sidx = nf[:, NFD:NFD + 1].astype(jnp.int32)
    oh_sh = (sidx == lax.broadcasted_iota(jnp.int32, (1, SET), 1)).astype(_f32)
    opc = opc_ref[...]
    oh_op = (opc == lax.broadcasted_iota(jnp.int32, (1, N_OPS), 1)).astype(_f32)
    t_sh = jnp.dot(emb_sh_ref[...], W1[NFD:NFD + SED],
                   preferred_element_type=_f32)
    t_op = jnp.dot(emb_op_ref[...], W1[NFD + SED:NFD + SED + OPD],
                   preferred_element_type=_f32)
    acc = jnp.dot(nf[:, :NFD], W1[:NFD], preferred_element_type=_f32)
    acc = acc + jnp.dot(ncf_ref[...], W1[NFD + SED + OPD:],
                        preferred_element_type=_f32)
    acc = acc + jnp.dot(oh_sh, t_sh, preferred_element_type=_f32)
    acc = acc + jnp.dot(oh_op, t_op, preferred_element_type=_f32)
    out_ref[...] = jnp.maximum(acc + b1_ref[...], 0.0)


def _pre_body(x_ref, pW_ref, pb_ref, Wl_ref, Wr_ref, y_ref, r_ref):
    x = x_ref[...]
    xp = jnp.maximum(jnp.dot(x, pW_ref[...], preferred_element_type=_f32)
                     + pb_ref[...], 0.0)
    y = jnp.dot(xp, Wl_ref[...], preferred_element_type=_f32)
    y_ref[0, :x.shape[0], :] = y[:, :HW]
    y_ref[1, :x.shape[0], :] = y[:, HW:]
    r_ref[...] = jnp.dot(x, Wr_ref[...], preferred_element_type=_f32)


def _post0_body(s_ref, cntp_ref, r_ref, bl_ref, x_ref, cnt_ref):
    cnt = lax.dot_general(cntp_ref[...], jnp.ones((NW, 1), _f32),
                          (((0,), (0,)), ((), ())),
                          preferred_element_type=_f32)
    cnt_ref[...] = cnt
    s = s_ref[...]
    z = jnp.concatenate([s[0], s[1]], axis=-1) / jnp.maximum(cnt, 1.0)
    out = z + bl_ref[...] + r_ref[...]
    nrm = jnp.sqrt(jnp.sum(out * out, axis=1, keepdims=True))
    x_ref[...] = out / jnp.maximum(nrm, 1e-12)


def _post_body(s_ref, cnt_ref, r_ref, bl_ref, x_ref):
    s = s_ref[...]
    z = jnp.concatenate([s[0], s[1]], axis=-1) / jnp.maximum(cnt_ref[...], 1.0)
    out = z + bl_ref[...] + r_ref[...]
    nrm = jnp.sqrt(jnp.sum(out * out, axis=1, keepdims=True))
    x_ref[...] = out / jnp.maximum(nrm, 1e-12)


def _pool_body(x_ref, batch_ref, pW_ref, pb_ref, out_ref):
    x = x_ref[...]
    b = batch_ref[...]
    oh = (b == lax.broadcasted_iota(jnp.int32, (1, G), 1)).astype(_f32)
    xsum = jax.lax.dot_general(oh, x, (((0,), (0,)), ((), ())),
                               preferred_element_type=_f32)
    gcnt = lax.dot_general(oh, jnp.ones((x.shape[0], 1), _f32),
                           (((0,), (0,)), ((), ())),
                           preferred_element_type=_f32)
    neg = jnp.float32(-jnp.inf)
    rows = []
    for g in range(G):
        m = jnp.max(jnp.where(oh[:, g:g + 1] > 0, x, neg), axis=0,
                    keepdims=True)
        rows.append(m)
    xmax = jnp.concatenate(rows, axis=0)
    xg = xmax + xsum / jnp.maximum(gcnt, 1.0)
    xg = xg / jnp.sqrt(jnp.sum(xg * xg, axis=1, keepdims=True))
    out_ref[...] = jnp.dot(xg, pW_ref[...], preferred_element_type=_f32) \
        + pb_ref[...]


def _tc_call(body, out_shapes):
    return pl.pallas_call(body, out_shape=out_shapes)


def kernel(node_feat, node_config_feat, node_opcode, edge_index, batch,
           emb_op, emb_shape, W1, b1,
           pW0, pb0, Wl0, bl0, Wr0,
           pW1, pb1, Wl1, bl1, Wr1,
           pW2, pb2, Wl2, bl2, Wr2,
           post_W, post_b):
    n = node_feat.shape[0]
    e = edge_index.shape[1]
    src = edge_index[0].astype(jnp.int32)
    dst = edge_index[1].astype(jnp.int32)
    e_pad = -(-e // (NS * CH * K)) * (NS * CH * K)
    pad = e_pad - e
    srcp = jnp.concatenate([src, jnp.zeros((pad,), jnp.int32)])
    dstp = jnp.concatenate([dst, jnp.full((pad,), n, jnp.int32)])

    opc = node_opcode.astype(jnp.int32).reshape(n, 1)
    bat = batch.astype(jnp.int32).reshape(n, 1)

    sds = jax.ShapeDtypeStruct
    x0 = _tc_call(_encode_body, sds((n, 2 * H), _f32))(
        node_feat, node_config_feat, opc, emb_op, emb_shape, W1,
        b1.reshape(1, -1))

    seg0 = _make_seg_kernel(e_pad, True)
    seg = _make_seg_kernel(e_pad, False)
    pre = _tc_call(_pre_body, [sds((NC, NPAD, HW), _f32), sds((n, H), _f32)])
    post0 = _tc_call(_post0_body, [sds((n, H), _f32), sds((n, 1), _f32)])
    post = _tc_call(_post_body, sds((n, H), _f32))

    y, r = pre(x0, pW0, pb0.reshape(1, -1), Wl0, Wr0)
    s, cntp = seg0(y, srcp, dstp)
    x, cnt = post0(s[:, :n], cntp[:, :n], r, bl0.reshape(1, -1))
    y, r = pre(x, pW1, pb1.reshape(1, -1), Wl1, Wr1)
    s = seg(y, srcp, dstp)
    x = post(s[:, :n], cnt, r, bl1.reshape(1, -1))
    y, r = pre(x, pW2, pb2.reshape(1, -1), Wl2, Wr2)
    s = seg(y, srcp, dstp)
    x = post(s[:, :n], cnt, r, bl2.reshape(1, -1))

    out = _tc_call(_pool_body, sds((G, 1), _f32))(
        x, bat, post_W, post_b.reshape(1, -1))
    return (out, out)
